# baseline (speedup 1.0000x reference)
import jax
import jax.numpy as jnp
from jax.experimental import pallas as pl

N = 16384
K = 512
BN = 2048


def _score_block(s_ref, p_ref, o_ref, out_ref):
    d = s_ref[0] + p_ref[0] - o_ref[0]
    out_ref[...] = -jnp.sqrt(jnp.sum(d * d, axis=1))


def kernel(triples):
    spec = lambda plane: pl.BlockSpec((1, BN, K), lambda i, p=plane: (p, i, 0))
    return pl.pallas_call(
        _score_block,
        grid=(N // BN,),
        in_specs=[spec(0), spec(1), spec(2)],
        out_specs=pl.BlockSpec((BN,), lambda i: (i,)),
        out_shape=jax.ShapeDtypeStruct((N,), jnp.float32),
    )(triples, triples, triples)


# speedup vs baseline: 1.0129x; 1.0129x over previous
import jax
import jax.numpy as jnp
from jax.experimental import pallas as pl

N = 16384
K = 512
BN = 2048


def _score_block(t_ref, o_ref):
    d = t_ref[0] + t_ref[1] - t_ref[2]
    o_ref[...] = -jnp.sqrt(jnp.sum(d * d, axis=1))


def kernel(triples):
    return pl.pallas_call(
        _score_block,
        grid=(N // BN,),
        in_specs=[pl.BlockSpec((3, BN, K), lambda i: (0, i, 0))],
        out_specs=pl.BlockSpec((BN,), lambda i: (i,)),
        out_shape=jax.ShapeDtypeStruct((N,), jnp.float32),
    )(triples)
